# Initial kernel scaffold; baseline (speedup 1.0000x reference)
import jax
import jax.numpy as jnp
from jax.experimental import pallas as pl


def kernel(x, edge_index, edge_attr, W1, b1, W2, b2):
    raise NotImplementedError("write your pallas kernel here")


# speedup vs baseline: 5.0014x; 5.0014x over previous
import functools

import jax
import jax.numpy as jnp
from jax import lax
from jax.experimental import pallas as pl
from jax.experimental.pallas import tpu as pltpu
from jax.experimental.pallas import tpu_sc as plsc

N = 10000
E = 320000
D = 128
H = 128
C = 64

NC = 2
NS = 16
NW = NC * NS
B = 80
NIT = 128
EB = NW * NIT
EPAD = EB * B
NIT2 = EB // NS
IB = 16
NBLK = NIT2 // IB
NBLKD = NIT // IB
NPAD = 10240
RPT = NPAD // NS
NHALF = 5120
NR = NHALF + 128
RPA = NR // NS


_MESH = plsc.VectorSubcoreMesh(core_axis_name="c", subcore_axis_name="s")


@functools.partial(
    pl.kernel,
    out_type=jax.ShapeDtypeStruct((NC, NPAD), jnp.float32),
    mesh=_MESH,
    scratch_types=[
        pltpu.VMEM((IB, B), jnp.int32),
        pltpu.VMEM((IB, B), jnp.float32),
        pltpu.VMEM((RPT,), jnp.float32),
        pltpu.VMEM_SHARED((NPAD,), jnp.float32),
    ],
)
def _sc_deg(row_hbm, w_hbm, out_hbm, row_v, w_v, zero_v, deg_sp):
    c = lax.axis_index("c")
    s = lax.axis_index("s")
    wid = s * NC + c
    base = wid * NIT
    for i in range(RPT // 16):
        zero_v[pl.ds(i * 16, 16)] = jnp.zeros((16,), jnp.float32)
    pltpu.sync_copy(zero_v, deg_sp.at[pl.ds(s * RPT, RPT)])
    plsc.subcore_barrier()

    def blk_body(blk, carry):
        pltpu.sync_copy(row_hbm.at[pl.ds(base + blk * IB, IB)], row_v)
        pltpu.sync_copy(w_hbm.at[pl.ds(base + blk * IB, IB)], w_v)

        def body(j, carry2):
            pltpu.sync_copy(w_v.at[j], deg_sp.at[row_v.at[j]], add=True)
            return carry2

        lax.fori_loop(0, IB, body, 0)
        return carry

    lax.fori_loop(0, NBLKD, blk_body, 0)
    plsc.subcore_barrier()
    pltpu.sync_copy(deg_sp.at[pl.ds(s * RPT, RPT)],
                    out_hbm.at[c, pl.ds(s * RPT, RPT)])


@functools.partial(
    pl.kernel,
    out_type=jax.ShapeDtypeStruct((NC, NR, H), jnp.float32),
    mesh=_MESH,
    scratch_types=[
        pltpu.VMEM((IB, B), jnp.int32),
        pltpu.VMEM((IB, B), jnp.int32),
        pltpu.VMEM((IB, B), jnp.int32),
        pltpu.VMEM((IB, B), jnp.float32),
        pltpu.VMEM((B, H), jnp.float32),
        pltpu.VMEM_SHARED((NR, H), jnp.float32),
        pltpu.SemaphoreType.DMA,
    ],
)
def _sc_agg(col_hbm, row_hbm, w_hbm, y_hbm, out_hbm,
            col_v, row_v, adj_v, w_v, rows_v, agg_sp, sem):
    c = lax.axis_index("c")
    s = lax.axis_index("s")
    base = s * NIT2
    chalf = c * NHALF

    def zb(i, carry):
        for k in range(H // 16):
            rows_v[i, pl.ds(k * 16, 16)] = jnp.zeros((16,), jnp.float32)
        return carry

    lax.fori_loop(0, B, zb, 0)
    for r in range(RPA // B):
        pltpu.sync_copy(rows_v, agg_sp.at[pl.ds(s * RPA + r * B, B)])
    pltpu.sync_copy(rows_v.at[pl.ds(0, RPA % B)],
                    agg_sp.at[pl.ds(s * RPA + (RPA // B) * B, RPA % B)])
    plsc.subcore_barrier()

    def blk_body(blk, carry):
        boff = base + blk * IB
        pltpu.sync_copy(col_hbm.at[pl.ds(boff, IB)], col_v)
        pltpu.sync_copy(row_hbm.at[pl.ds(boff, IB)], row_v)
        pltpu.sync_copy(w_hbm.at[pl.ds(boff, IB)], w_v)

        def body(j, carry2):
            pltpu.async_copy(y_hbm.at[col_v.at[j]], rows_v, sem).wait()

            def scale(g, carry3):
                wv = w_v[j, pl.ds(g * 16, 16)]
                rv = row_v[j, pl.ds(g * 16, 16)]
                t = rv - chalf
                ok = (t >= 0) & (t < NHALF)
                adj_v[j, pl.ds(g * 16, 16)] = jnp.where(ok, t, NHALF)
                for l in range(16):
                    ws = wv[l]
                    i = g * 16 + l
                    for k in range(H // 16):
                        sl = pl.ds(k * 16, 16)
                        rows_v[i, sl] = rows_v[i, sl] * ws
                return carry3

            lax.fori_loop(0, B // 16, scale, 0)
            pltpu.sync_copy(rows_v, agg_sp.at[adj_v.at[j]], add=True)
            return carry2

        lax.fori_loop(0, IB, body, 0)
        return carry

    lax.fori_loop(0, NBLK, blk_body, 0)
    plsc.subcore_barrier()
    pltpu.sync_copy(agg_sp.at[pl.ds(s * RPA, RPA)],
                    out_hbm.at[c, pl.ds(s * RPA, RPA)])


def _tc1_body(x_ref, w1_ref, dinv_ref, y_ref):
    xw = lax.dot_general(x_ref[...], w1_ref[...], (((1,), (0,)), ((), ())),
                         precision=lax.Precision.HIGHEST,
                         preferred_element_type=jnp.float32)
    y_ref[...] = xw * dinv_ref[...]


def _tc2_body(aggp_ref, y1_ref, dinv_ref, b1_ref, w2_ref, y2_ref):
    agg = jnp.concatenate([aggp_ref[0, :NHALF, :],
                           aggp_ref[1, :N - NHALF, :]], axis=0)
    pre = (agg + y1_ref[...]) * dinv_ref[...] + b1_ref[...]
    h = jnp.maximum(pre, 0.0)
    y2 = lax.dot_general(h, w2_ref[...], (((1,), (0,)), ((), ())),
                         precision=lax.Precision.HIGHEST,
                         preferred_element_type=jnp.float32) * dinv_ref[...]
    y2_ref[...] = jnp.concatenate([y2, jnp.zeros_like(y2)], axis=1)


def _tc3_body(aggp_ref, y2p_ref, dinv_ref, b2_ref, fin_ref, ls_ref):
    agg = jnp.concatenate([aggp_ref[0, :NHALF, :C],
                           aggp_ref[1, :N - NHALF, :C]], axis=0)
    y2 = y2p_ref[:, :C]
    final = (agg + y2) * dinv_ref[...] + b2_ref[...]
    m = jnp.max(final, axis=1, keepdims=True)
    lse = m + jnp.log(jnp.sum(jnp.exp(final - m), axis=1, keepdims=True))
    fin_ref[...] = final
    ls_ref[...] = final - lse


def kernel(x, edge_index, edge_attr, W1, b1, W2, b2):
    assert x.shape == (N, D) and edge_index.shape == (2, E)
    pad = EPAD - E
    zpad_i = jnp.zeros((pad,), jnp.int32)
    row2 = jnp.concatenate([edge_index[0], zpad_i]).reshape(EB, B)
    col2 = jnp.concatenate([edge_index[1], zpad_i]).reshape(EB, B)
    w2e = jnp.concatenate([edge_attr, jnp.zeros((pad,), jnp.float32)]).reshape(EB, B)

    degp = _sc_deg(row2, w2e)
    deg = degp[0, :N] + degp[1, :N] + 1.0
    dinv = jnp.where(deg > 0, lax.rsqrt(jnp.where(deg > 0, deg, 1.0)), 0.0)
    dinv = dinv[:, None]

    y1 = pl.pallas_call(
        _tc1_body,
        out_shape=jax.ShapeDtypeStruct((N, H), jnp.float32),
    )(x, W1, dinv)

    agg1p = _sc_agg(col2, row2, w2e, y1)

    y2p = pl.pallas_call(
        _tc2_body,
        out_shape=jax.ShapeDtypeStruct((N, 2 * C), jnp.float32),
    )(agg1p, y1, dinv, b1.reshape(1, H), W2)

    agg2p = _sc_agg(col2, row2, w2e, y2p)

    final, ls = pl.pallas_call(
        _tc3_body,
        out_shape=(jax.ShapeDtypeStruct((N, C), jnp.float32),
                   jax.ShapeDtypeStruct((N, C), jnp.float32)),
    )(agg2p, y2p, dinv, b2.reshape(1, C))

    return (final, ls)


# speedup vs baseline: 5.6855x; 1.1368x over previous
import functools

import jax
import jax.numpy as jnp
from jax import lax
from jax.experimental import pallas as pl
from jax.experimental.pallas import tpu as pltpu
from jax.experimental.pallas import tpu_sc as plsc

N = 10000
E = 320000
D = 128
H = 128
C = 64

NC = 2
NS = 16
NW = NC * NS
B = 80
NIT = 128
EB = NW * NIT
EPAD = EB * B
NIT2 = EB // NS
IB = 16
NBLK = NIT2 // IB
NBLKD = NIT // IB
NPAD = 10240
RPT = NPAD // NS
NHALF = 5120
NR = NHALF + 128
RPA = NR // NS


_MESH = plsc.VectorSubcoreMesh(core_axis_name="c", subcore_axis_name="s")


@functools.partial(
    pl.kernel,
    out_type=jax.ShapeDtypeStruct((NC, NPAD), jnp.float32),
    mesh=_MESH,
    scratch_types=[
        pltpu.VMEM((IB, B), jnp.int32),
        pltpu.VMEM((IB, B), jnp.float32),
        pltpu.VMEM((RPT,), jnp.float32),
        pltpu.VMEM_SHARED((NPAD,), jnp.float32),
    ],
)
def _sc_deg(row_hbm, w_hbm, out_hbm, row_v, w_v, zero_v, deg_sp):
    c = lax.axis_index("c")
    s = lax.axis_index("s")
    wid = s * NC + c
    base = wid * NIT
    for i in range(RPT // 16):
        zero_v[pl.ds(i * 16, 16)] = jnp.zeros((16,), jnp.float32)
    pltpu.sync_copy(zero_v, deg_sp.at[pl.ds(s * RPT, RPT)])
    plsc.subcore_barrier()

    def blk_body(blk, carry):
        pltpu.sync_copy(row_hbm.at[pl.ds(base + blk * IB, IB)], row_v)
        pltpu.sync_copy(w_hbm.at[pl.ds(base + blk * IB, IB)], w_v)

        def body(j, carry2):
            pltpu.sync_copy(w_v.at[j], deg_sp.at[row_v.at[j]], add=True)
            return carry2

        lax.fori_loop(0, IB, body, 0)
        return carry

    lax.fori_loop(0, NBLKD, blk_body, 0)
    plsc.subcore_barrier()
    pltpu.sync_copy(deg_sp.at[pl.ds(s * RPT, RPT)],
                    out_hbm.at[c, pl.ds(s * RPT, RPT)])


BUFB = B * H * 4


@functools.partial(
    pl.kernel,
    out_type=jax.ShapeDtypeStruct((NC, NR, H), jnp.float32),
    mesh=_MESH,
    scratch_types=[
        pltpu.VMEM((IB, B), jnp.int32),
        pltpu.VMEM((IB, B), jnp.int32),
        pltpu.VMEM((IB, B), jnp.int32),
        pltpu.VMEM((IB, B), jnp.float32),
        pltpu.VMEM((B, H), jnp.float32),
        pltpu.VMEM((B, H), jnp.float32),
        pltpu.VMEM_SHARED((NR, H), jnp.float32),
        pltpu.SemaphoreType.DMA,
        pltpu.SemaphoreType.DMA,
        pltpu.SemaphoreType.DMA,
        pltpu.SemaphoreType.DMA,
    ],
)
def _sc_agg(col_hbm, row_hbm, w_hbm, y_hbm, out_hbm,
            col_v, row_v, adj_v, w_v, buf0, buf1, agg_sp,
            gsem0, gsem1, ssem0, ssem1):
    c = lax.axis_index("c")
    s = lax.axis_index("s")
    base = s * NIT2
    chalf = c * NHALF
    bufs = (buf0, buf1)
    gsems = (gsem0, gsem1)
    ssems = (ssem0, ssem1)

    def zb(i, carry):
        for k in range(H // 16):
            buf0[i, pl.ds(k * 16, 16)] = jnp.zeros((16,), jnp.float32)
        return carry

    lax.fori_loop(0, B, zb, 0)
    for r in range(RPA // B):
        pltpu.sync_copy(buf0, agg_sp.at[pl.ds(s * RPA + r * B, B)])
    pltpu.sync_copy(buf0.at[pl.ds(0, RPA % B)],
                    agg_sp.at[pl.ds(s * RPA + (RPA // B) * B, RPA % B)])
    pltpu.async_copy(y_hbm.at[pl.ds(0, B)], buf0, ssem0)
    pltpu.async_copy(y_hbm.at[pl.ds(0, B)], buf1, ssem1)
    plsc.subcore_barrier()


    def wait_sem(sem, buf):
        pltpu.make_async_copy(y_hbm.at[pl.ds(0, B)], buf, sem).wait()

    def scale_and_scatter(jj, p):
        buf = bufs[p]

        def scale(g, carry):
            wv = w_v[jj, pl.ds(g * 16, 16)]
            rv = row_v[jj, pl.ds(g * 16, 16)]
            t = rv - chalf
            ok = (t >= 0) & (t < NHALF)
            adj_v[jj, pl.ds(g * 16, 16)] = jnp.where(ok, t, NHALF)
            for l in range(16):
                ws = wv[l]
                i = g * 16 + l
                for k in range(H // 16):
                    sl = pl.ds(k * 16, 16)
                    buf[i, sl] = buf[i, sl] * ws
            return carry

        lax.fori_loop(0, B // 16, scale, 0)
        pltpu.async_copy(buf, agg_sp.at[adj_v.at[jj]], ssems[p], add=True)

    def blk_body(blk, carry):
        boff = base + blk * IB
        pltpu.sync_copy(col_hbm.at[pl.ds(boff, IB)], col_v)
        pltpu.sync_copy(row_hbm.at[pl.ds(boff, IB)], row_v)
        pltpu.sync_copy(w_hbm.at[pl.ds(boff, IB)], w_v)
        wait_sem(ssem0, buf0)
        pltpu.async_copy(y_hbm.at[col_v.at[0]], buf0, gsem0)

        def pair_body(g2, carry2):
            for p in (0, 1):
                jj = g2 * 2 + p
                wait_sem(gsems[p], bufs[p])
                if p == 0:
                    wait_sem(ssems[1], buf1)
                    pltpu.async_copy(y_hbm.at[col_v.at[jj + 1]], buf1, gsem1)
                else:
                    @pl.when(g2 < IB // 2 - 1)
                    def _():
                        wait_sem(ssems[0], buf0)
                        pltpu.async_copy(y_hbm.at[col_v.at[jj + 1]], buf0,
                                         gsem0)
                scale_and_scatter(jj, p)
            return carry2

        lax.fori_loop(0, IB // 2, pair_body, 0)
        return carry

    lax.fori_loop(0, NBLK, blk_body, 0)
    wait_sem(ssem0, buf0)
    wait_sem(ssem1, buf1)
    plsc.subcore_barrier()
    pltpu.sync_copy(agg_sp.at[pl.ds(s * RPA, RPA)],
                    out_hbm.at[c, pl.ds(s * RPA, RPA)])


def _tc1_body(x_ref, w1_ref, dinv_ref, y_ref):
    xw = lax.dot_general(x_ref[...], w1_ref[...], (((1,), (0,)), ((), ())),
                         precision=lax.Precision.HIGHEST,
                         preferred_element_type=jnp.float32)
    y_ref[...] = xw * dinv_ref[...]


def _tc2_body(aggp_ref, y1_ref, dinv_ref, b1_ref, w2_ref, y2_ref):
    agg = jnp.concatenate([aggp_ref[0, :NHALF, :],
                           aggp_ref[1, :N - NHALF, :]], axis=0)
    pre = (agg + y1_ref[...]) * dinv_ref[...] + b1_ref[...]
    h = jnp.maximum(pre, 0.0)
    y2 = lax.dot_general(h, w2_ref[...], (((1,), (0,)), ((), ())),
                         precision=lax.Precision.HIGHEST,
                         preferred_element_type=jnp.float32) * dinv_ref[...]
    y2_ref[...] = jnp.concatenate([y2, jnp.zeros_like(y2)], axis=1)


def _tc3_body(aggp_ref, y2p_ref, dinv_ref, b2_ref, fin_ref, ls_ref):
    agg = jnp.concatenate([aggp_ref[0, :NHALF, :C],
                           aggp_ref[1, :N - NHALF, :C]], axis=0)
    y2 = y2p_ref[:, :C]
    final = (agg + y2) * dinv_ref[...] + b2_ref[...]
    m = jnp.max(final, axis=1, keepdims=True)
    lse = m + jnp.log(jnp.sum(jnp.exp(final - m), axis=1, keepdims=True))
    fin_ref[...] = final
    ls_ref[...] = final - lse


def kernel(x, edge_index, edge_attr, W1, b1, W2, b2):
    assert x.shape == (N, D) and edge_index.shape == (2, E)
    pad = EPAD - E
    zpad_i = jnp.zeros((pad,), jnp.int32)
    row2 = jnp.concatenate([edge_index[0], zpad_i]).reshape(EB, B)
    col2 = jnp.concatenate([edge_index[1], zpad_i]).reshape(EB, B)
    w2e = jnp.concatenate([edge_attr, jnp.zeros((pad,), jnp.float32)]).reshape(EB, B)

    degp = _sc_deg(row2, w2e)
    deg = degp[0, :N] + degp[1, :N] + 1.0
    dinv = jnp.where(deg > 0, lax.rsqrt(jnp.where(deg > 0, deg, 1.0)), 0.0)
    dinv = dinv[:, None]

    y1 = pl.pallas_call(
        _tc1_body,
        out_shape=jax.ShapeDtypeStruct((N, H), jnp.float32),
    )(x, W1, dinv)

    agg1p = _sc_agg(col2, row2, w2e, y1)

    y2p = pl.pallas_call(
        _tc2_body,
        out_shape=jax.ShapeDtypeStruct((N, 2 * C), jnp.float32),
    )(agg1p, y1, dinv, b1.reshape(1, H), W2)

    agg2p = _sc_agg(col2, row2, w2e, y2p)

    final, ls = pl.pallas_call(
        _tc3_body,
        out_shape=(jax.ShapeDtypeStruct((N, C), jnp.float32),
                   jax.ShapeDtypeStruct((N, C), jnp.float32)),
    )(agg2p, y2p, dinv, b2.reshape(1, C))

    return (final, ls)


# speedup vs baseline: 5.7355x; 1.0088x over previous
import functools

import jax
import jax.numpy as jnp
from jax import lax
from jax.experimental import pallas as pl
from jax.experimental.pallas import tpu as pltpu
from jax.experimental.pallas import tpu_sc as plsc

N = 10000
E = 320000
D = 128
H = 128
C = 64

NC = 2
NS = 16
NW = NC * NS
B = 80
NIT = 128
EB = NW * NIT
EPAD = EB * B
NIT2 = EB // NS
IB = 16
NBLK = NIT2 // IB
NBLKD = NIT // IB
NPAD = 10240
RPT = NPAD // NS
NHALF = 5120
NR = NHALF + 128
RPA = NR // NS


_MESH = plsc.VectorSubcoreMesh(core_axis_name="c", subcore_axis_name="s")


@functools.partial(
    pl.kernel,
    out_type=jax.ShapeDtypeStruct((NC, NPAD), jnp.float32),
    mesh=_MESH,
    scratch_types=[
        pltpu.VMEM((IB, B), jnp.int32),
        pltpu.VMEM((IB, B), jnp.float32),
        pltpu.VMEM((RPT,), jnp.float32),
        pltpu.VMEM_SHARED((NPAD,), jnp.float32),
    ],
)
def _sc_deg(row_hbm, w_hbm, out_hbm, row_v, w_v, zero_v, deg_sp):
    c = lax.axis_index("c")
    s = lax.axis_index("s")
    wid = s * NC + c
    base = wid * NIT
    for i in range(RPT // 16):
        zero_v[pl.ds(i * 16, 16)] = jnp.zeros((16,), jnp.float32)
    pltpu.sync_copy(zero_v, deg_sp.at[pl.ds(s * RPT, RPT)])
    plsc.subcore_barrier()

    def blk_body(blk, carry):
        pltpu.sync_copy(row_hbm.at[pl.ds(base + blk * IB, IB)], row_v)
        pltpu.sync_copy(w_hbm.at[pl.ds(base + blk * IB, IB)], w_v)

        def body(j, carry2):
            pltpu.sync_copy(w_v.at[j], deg_sp.at[row_v.at[j]], add=True)
            return carry2

        lax.fori_loop(0, IB, body, 0)
        return carry

    lax.fori_loop(0, NBLKD, blk_body, 0)
    plsc.subcore_barrier()
    pltpu.sync_copy(deg_sp.at[pl.ds(s * RPT, RPT)],
                    out_hbm.at[c, pl.ds(s * RPT, RPT)])


BUFB = B * H * 4


@functools.partial(
    pl.kernel,
    out_type=jax.ShapeDtypeStruct((NC, NR, H), jnp.float32),
    mesh=_MESH,
    scratch_types=[
        pltpu.VMEM((IB, B), jnp.int32),
        pltpu.VMEM((IB, B), jnp.int32),
        pltpu.VMEM((IB, B), jnp.int32),
        pltpu.VMEM((IB, B), jnp.float32),
        pltpu.VMEM((B, H), jnp.float32),
        pltpu.VMEM((B, H), jnp.float32),
        pltpu.VMEM_SHARED((NR, H), jnp.float32),
        pltpu.SemaphoreType.DMA,
        pltpu.SemaphoreType.DMA,
        pltpu.SemaphoreType.DMA,
        pltpu.SemaphoreType.DMA,
    ],
)
def _sc_agg(col_hbm, row_hbm, w_hbm, y_hbm, out_hbm,
            col_v, row_v, adj_v, w_v, buf0, buf1, agg_sp,
            gsem0, gsem1, ssem0, ssem1):
    c = lax.axis_index("c")
    s = lax.axis_index("s")
    base = s * NIT2
    chalf = c * NHALF
    bufs = (buf0, buf1)
    gsems = (gsem0, gsem1)
    ssems = (ssem0, ssem1)

    def zb(i, carry):
        for k in range(H // 16):
            buf0[i, pl.ds(k * 16, 16)] = jnp.zeros((16,), jnp.float32)
        return carry

    lax.fori_loop(0, B, zb, 0)
    for r in range(RPA // B):
        pltpu.sync_copy(buf0, agg_sp.at[pl.ds(s * RPA + r * B, B)])
    pltpu.sync_copy(buf0.at[pl.ds(0, RPA % B)],
                    agg_sp.at[pl.ds(s * RPA + (RPA // B) * B, RPA % B)])
    pltpu.async_copy(y_hbm.at[pl.ds(0, B)], buf0, ssem0)
    pltpu.async_copy(y_hbm.at[pl.ds(0, B)], buf1, ssem1)
    plsc.subcore_barrier()


    def wait_sem(sem, buf):
        pltpu.make_async_copy(y_hbm.at[pl.ds(0, B)], buf, sem).wait()

    def scale_and_scatter(jj, p):
        buf = bufs[p]

        def scale(g, carry):
            wv = w_v[jj, pl.ds(g * 16, 16)]
            rv = row_v[jj, pl.ds(g * 16, 16)]
            t = rv - chalf
            ok = (t >= 0) & (t < NHALF)
            dump = NHALF + s * 8 + (lax.iota(jnp.int32, 16) & 7)
            adj_v[jj, pl.ds(g * 16, 16)] = jnp.where(ok, t, dump)
            for l in range(16):
                ws = wv[l]
                i = g * 16 + l
                for k in range(H // 16):
                    sl = pl.ds(k * 16, 16)
                    buf[i, sl] = buf[i, sl] * ws
            return carry

        lax.fori_loop(0, B // 16, scale, 0)
        pltpu.async_copy(buf, agg_sp.at[adj_v.at[jj]], ssems[p], add=True)

    def blk_body(blk, carry):
        boff = base + blk * IB
        pltpu.sync_copy(col_hbm.at[pl.ds(boff, IB)], col_v)
        pltpu.sync_copy(row_hbm.at[pl.ds(boff, IB)], row_v)
        pltpu.sync_copy(w_hbm.at[pl.ds(boff, IB)], w_v)
        wait_sem(ssem0, buf0)
        pltpu.async_copy(y_hbm.at[col_v.at[0]], buf0, gsem0)

        def pair_body(g2, carry2):
            for p in (0, 1):
                jj = g2 * 2 + p
                wait_sem(gsems[p], bufs[p])
                if p == 0:
                    wait_sem(ssems[1], buf1)
                    pltpu.async_copy(y_hbm.at[col_v.at[jj + 1]], buf1, gsem1)
                else:
                    @pl.when(g2 < IB // 2 - 1)
                    def _():
                        wait_sem(ssems[0], buf0)
                        pltpu.async_copy(y_hbm.at[col_v.at[jj + 1]], buf0,
                                         gsem0)
                scale_and_scatter(jj, p)
            return carry2

        lax.fori_loop(0, IB // 2, pair_body, 0)
        return carry

    lax.fori_loop(0, NBLK, blk_body, 0)
    wait_sem(ssem0, buf0)
    wait_sem(ssem1, buf1)
    plsc.subcore_barrier()
    pltpu.sync_copy(agg_sp.at[pl.ds(s * RPA, RPA)],
                    out_hbm.at[c, pl.ds(s * RPA, RPA)])


def _tc1_body(x_ref, w1_ref, dinv_ref, y_ref):
    xw = lax.dot_general(x_ref[...], w1_ref[...], (((1,), (0,)), ((), ())),
                         precision=lax.Precision.HIGHEST,
                         preferred_element_type=jnp.float32)
    y_ref[...] = xw * dinv_ref[...]


def _tc2_body(aggp_ref, y1_ref, dinv_ref, b1_ref, w2_ref, y2_ref):
    agg = jnp.concatenate([aggp_ref[0, :NHALF, :],
                           aggp_ref[1, :N - NHALF, :]], axis=0)
    pre = (agg + y1_ref[...]) * dinv_ref[...] + b1_ref[...]
    h = jnp.maximum(pre, 0.0)
    y2 = lax.dot_general(h, w2_ref[...], (((1,), (0,)), ((), ())),
                         precision=lax.Precision.HIGHEST,
                         preferred_element_type=jnp.float32) * dinv_ref[...]
    y2_ref[...] = jnp.concatenate([y2, jnp.zeros_like(y2)], axis=1)


def _tc3_body(aggp_ref, y2p_ref, dinv_ref, b2_ref, fin_ref, ls_ref):
    agg = jnp.concatenate([aggp_ref[0, :NHALF, :C],
                           aggp_ref[1, :N - NHALF, :C]], axis=0)
    y2 = y2p_ref[:, :C]
    final = (agg + y2) * dinv_ref[...] + b2_ref[...]
    m = jnp.max(final, axis=1, keepdims=True)
    lse = m + jnp.log(jnp.sum(jnp.exp(final - m), axis=1, keepdims=True))
    fin_ref[...] = final
    ls_ref[...] = final - lse


def kernel(x, edge_index, edge_attr, W1, b1, W2, b2):
    assert x.shape == (N, D) and edge_index.shape == (2, E)
    pad = EPAD - E
    zpad_i = jnp.zeros((pad,), jnp.int32)
    row2 = jnp.concatenate([edge_index[0], zpad_i]).reshape(EB, B)
    col2 = jnp.concatenate([edge_index[1], zpad_i]).reshape(EB, B)
    w2e = jnp.concatenate([edge_attr, jnp.zeros((pad,), jnp.float32)]).reshape(EB, B)

    degp = _sc_deg(row2, w2e)
    deg = degp[0, :N] + degp[1, :N] + 1.0
    dinv = jnp.where(deg > 0, lax.rsqrt(jnp.where(deg > 0, deg, 1.0)), 0.0)
    dinv = dinv[:, None]

    y1 = pl.pallas_call(
        _tc1_body,
        out_shape=jax.ShapeDtypeStruct((N, H), jnp.float32),
    )(x, W1, dinv)

    agg1p = _sc_agg(col2, row2, w2e, y1)

    y2p = pl.pallas_call(
        _tc2_body,
        out_shape=jax.ShapeDtypeStruct((N, 2 * C), jnp.float32),
    )(agg1p, y1, dinv, b1.reshape(1, H), W2)

    agg2p = _sc_agg(col2, row2, w2e, y2p)

    final, ls = pl.pallas_call(
        _tc3_body,
        out_shape=(jax.ShapeDtypeStruct((N, C), jnp.float32),
                   jax.ShapeDtypeStruct((N, C), jnp.float32)),
    )(agg2p, y2p, dinv, b2.reshape(1, C))

    return (final, ls)
